# Initial kernel scaffold; baseline (speedup 1.0000x reference)
import jax
import jax.numpy as jnp
from jax.experimental import pallas as pl


def kernel(x, emb_weight):
    raise NotImplementedError("write your pallas kernel here")


# speedup vs baseline: 3.0944x; 3.0944x over previous
import jax
import jax.numpy as jnp
from jax.experimental import pallas as pl
from jax.experimental.pallas import tpu as pltpu
from jax.experimental.pallas import tpu_sc as plsc

EMB_DIM = 128
WINDOW = 128

_vector_mesh = plsc.VectorSubcoreMesh(
    core_axis_name="core", subcore_axis_name="subcore"
)


def _gather_rows(table, idx_flat):
    n_idx = idx_flat.shape[0]
    idx2d = idx_flat.reshape(1, n_idx)

    @pl.kernel(
        out_type=jax.ShapeDtypeStruct((n_idx, EMB_DIM), table.dtype),
        mesh=_vector_mesh,
    )
    def _kernel(table_hbm, idx_hbm, out_hbm):
        def body(i_vmem, o_vmem):
            pltpu.sync_copy(table_hbm.at[i_vmem.at[0]], o_vmem)

        pltpu.emit_pipeline(
            body,
            grid=(n_idx // WINDOW,),
            in_specs=[pl.BlockSpec((1, WINDOW), index_map=lambda i: (0, i))],
            out_specs=[
                pl.BlockSpec((WINDOW, EMB_DIM), index_map=lambda i: (i, 0))
            ],
            core_axis_name=("core", "subcore"),
            dimension_semantics=(pltpu.PARALLEL,),
        )(idx_hbm, out_hbm)

    return _kernel(table, idx2d)


def kernel(x, emb_weight):
    batch, hist = x.shape
    idx_flat = x.reshape(-1).astype(jnp.int32)
    rows = _gather_rows(emb_weight, idx_flat)
    return rows.reshape(batch, hist, EMB_DIM)


# speedup vs baseline: 3.3469x; 1.0816x over previous
import jax
import jax.numpy as jnp
from jax import lax
from jax.experimental import pallas as pl
from jax.experimental.pallas import tpu as pltpu
from jax.experimental.pallas import tpu_sc as plsc

EMB_DIM = 128
WINDOW = 128
NBUF = 5
LEAD = 3

_vector_mesh = plsc.VectorSubcoreMesh(
    core_axis_name="core", subcore_axis_name="subcore"
)
_NW = 32


def _gather_rows(table, idx3d):
    nchunk = idx3d.shape[1]
    n_idx = _NW * nchunk * WINDOW

    @pl.kernel(
        out_type=jax.ShapeDtypeStruct((n_idx, EMB_DIM), table.dtype),
        mesh=_vector_mesh,
        scratch_types=[
            pltpu.VMEM((nchunk, WINDOW), jnp.int32),
            pltpu.VMEM((NBUF, WINDOW, EMB_DIM), table.dtype),
            pltpu.SemaphoreType.DMA((NBUF,)),
            pltpu.SemaphoreType.DMA((NBUF,)),
        ],
    )
    def _kernel(table_hbm, idx_hbm, out_hbm, idx_v, bufs, gsem, osem):
        wid = lax.axis_index("subcore") * 2 + lax.axis_index("core")
        row_base = wid * (nchunk * WINDOW)

        def g_start(k, b):
            pltpu.make_async_copy(
                table_hbm.at[idx_v.at[k]], bufs.at[b], gsem.at[b]
            ).start()

        def g_wait(b):
            pltpu.make_async_copy(
                table_hbm.at[idx_v.at[0]], bufs.at[b], gsem.at[b]
            ).wait()

        def o_start(k, b):
            pltpu.make_async_copy(
                bufs.at[b],
                out_hbm.at[pl.ds(row_base + k * WINDOW, WINDOW)],
                osem.at[b],
            ).start()

        def o_wait(b):
            pltpu.make_async_copy(
                bufs.at[b], out_hbm.at[pl.ds(0, WINDOW)], osem.at[b]
            ).wait()

        pltpu.sync_copy(idx_hbm.at[wid], idx_v)

        for b in range(LEAD):
            g_start(b, b)

        for k in (0, 1):
            g_wait(k % NBUF)
            o_start(k, k % NBUF)
            g_start(k + LEAD, (k + LEAD) % NBUF)

        ngroups = (nchunk - LEAD - 2) // NBUF

        @pl.loop(0, ngroups)
        def _(g):
            for b in range(NBUF):
                k = g * NBUF + b + 2
                o_wait(b)
                g_start(k + LEAD, b)
                g_wait((b + 2) % NBUF)
                o_start(k, (b + 2) % NBUF)

        for k in range(ngroups * NBUF + 2, nchunk):
            b = k % NBUF
            g_wait(b)
            o_start(k, b)

        for b in range(NBUF):
            o_wait(b)

    return _kernel(table, idx3d)


def kernel(x, emb_weight):
    batch, hist = x.shape
    idx3d = x.reshape(_NW, -1, WINDOW).astype(jnp.int32)
    rows = _gather_rows(emb_weight, idx3d)
    return rows.reshape(batch, hist, EMB_DIM)


# speedup vs baseline: 5.9724x; 1.7845x over previous
import jax
import jax.numpy as jnp
from jax import lax
from jax.experimental import pallas as pl
from jax.experimental.pallas import tpu as pltpu
from jax.experimental.pallas import tpu_sc as plsc

EMB_DIM = 128
HIST = 50
BPW = 2
WINDOW = BPW * HIST
NBUF = 5
LEAD = 3

_vector_mesh = plsc.VectorSubcoreMesh(
    core_axis_name="core", subcore_axis_name="subcore"
)
_NW = 32


def _gather_rows(table, idx3d, batch):
    nchunk = idx3d.shape[1]

    @pl.kernel(
        out_type=jax.ShapeDtypeStruct((batch, HIST, EMB_DIM), table.dtype),
        mesh=_vector_mesh,
        scratch_types=[
            pltpu.VMEM((nchunk, WINDOW), jnp.int32),
            pltpu.VMEM((NBUF, WINDOW, EMB_DIM), table.dtype),
            pltpu.SemaphoreType.DMA((NBUF,)),
            pltpu.SemaphoreType.DMA((NBUF,)),
        ],
    )
    def _kernel(table_hbm, idx_hbm, out_hbm, idx_v, bufs, gsem, osem):
        wid = lax.axis_index("subcore") * 2 + lax.axis_index("core")
        elem_base = wid * (nchunk * BPW)

        def g_start(k, b):
            pltpu.make_async_copy(
                table_hbm.at[idx_v.at[k]], bufs.at[b], gsem.at[b]
            ).start()

        def g_wait(b):
            pltpu.make_async_copy(
                table_hbm.at[idx_v.at[0]], bufs.at[b], gsem.at[b]
            ).wait()

        def o_start(k, b):
            for e in range(BPW):
                pltpu.make_async_copy(
                    bufs.at[b].at[pl.ds(e * HIST, HIST)],
                    out_hbm.at[elem_base + k * BPW + e],
                    osem.at[b],
                ).start()

        def o_wait(b):
            for e in range(BPW):
                pltpu.make_async_copy(
                    bufs.at[b].at[pl.ds(e * HIST, HIST)],
                    out_hbm.at[0],
                    osem.at[b],
                ).wait()

        pltpu.sync_copy(idx_hbm.at[wid], idx_v)

        def step(k, kmod):
            if k + LEAD < nchunk:
                if k >= 2:
                    o_wait((kmod + LEAD) % NBUF)
                g_start(k + LEAD, (kmod + LEAD) % NBUF)
            g_wait(kmod % NBUF)
            o_start(k, kmod % NBUF)

        for b in range(LEAD):
            g_start(b, b)

        step(0, 0)
        step(1, 1)

        ngroups = (nchunk - LEAD - 2) // NBUF

        @pl.loop(0, ngroups)
        def _(g):
            for b in range(NBUF):
                k = g * NBUF + b + 2
                o_wait((b + 2 + LEAD) % NBUF)
                g_start(k + LEAD, (b + 2 + LEAD) % NBUF)
                g_wait((b + 2) % NBUF)
                o_start(k, (b + 2) % NBUF)

        for k in range(ngroups * NBUF + 2, nchunk):
            step(k, k % NBUF)

        for b in range(NBUF):
            o_wait(b)

    return _kernel(table, idx3d)


def kernel(x, emb_weight):
    batch, hist = x.shape
    idx3d = x.reshape(_NW, -1, WINDOW).astype(jnp.int32)
    return _gather_rows(emb_weight, idx3d, batch)


# speedup vs baseline: 5.9735x; 1.0002x over previous
import jax
import jax.numpy as jnp
from jax import lax
from jax.experimental import pallas as pl
from jax.experimental.pallas import tpu as pltpu
from jax.experimental.pallas import tpu_sc as plsc

EMB_DIM = 128
HIST = 50
BPW = 2
WINDOW = BPW * HIST
NBUF = 5
LEAD = 3

_vector_mesh = plsc.VectorSubcoreMesh(
    core_axis_name="core", subcore_axis_name="subcore"
)
_NW = 32


def _gather_rows(table, idx3d, batch):
    nchunk = idx3d.shape[1]

    @pl.kernel(
        out_type=jax.ShapeDtypeStruct((batch, HIST, EMB_DIM), table.dtype),
        mesh=_vector_mesh,
        compiler_params=pltpu.CompilerParams(use_tc_tiling_on_sc=True),
        scratch_types=[
            pltpu.VMEM((nchunk, WINDOW), jnp.int32),
            pltpu.VMEM((NBUF, WINDOW, EMB_DIM), table.dtype),
            pltpu.SemaphoreType.DMA((NBUF,)),
            pltpu.SemaphoreType.DMA((NBUF,)),
        ],
    )
    def _kernel(table_hbm, idx_hbm, out_hbm, idx_v, bufs, gsem, osem):
        wid = lax.axis_index("subcore") * 2 + lax.axis_index("core")
        elem_base = wid * (nchunk * BPW)

        def g_start(k, b):
            pltpu.make_async_copy(
                table_hbm.at[idx_v.at[k]], bufs.at[b], gsem.at[b]
            ).start()

        def g_wait(b):
            pltpu.make_async_copy(
                table_hbm.at[idx_v.at[0]], bufs.at[b], gsem.at[b]
            ).wait()

        def o_start(k, b):
            for e in range(BPW):
                pltpu.make_async_copy(
                    bufs.at[b].at[pl.ds(e * HIST, HIST)],
                    out_hbm.at[elem_base + k * BPW + e],
                    osem.at[b],
                ).start()

        def o_wait(b):
            for e in range(BPW):
                pltpu.make_async_copy(
                    bufs.at[b].at[pl.ds(e * HIST, HIST)],
                    out_hbm.at[0],
                    osem.at[b],
                ).wait()

        pltpu.sync_copy(idx_hbm.at[wid], idx_v)

        def step(k, kmod):
            if k + LEAD < nchunk:
                if k >= 2:
                    o_wait((kmod + LEAD) % NBUF)
                g_start(k + LEAD, (kmod + LEAD) % NBUF)
            g_wait(kmod % NBUF)
            o_start(k, kmod % NBUF)

        for b in range(LEAD):
            g_start(b, b)

        step(0, 0)
        step(1, 1)

        ngroups = (nchunk - LEAD - 2) // NBUF

        @pl.loop(0, ngroups)
        def _(g):
            for b in range(NBUF):
                k = g * NBUF + b + 2
                o_wait((b + 2 + LEAD) % NBUF)
                g_start(k + LEAD, (b + 2 + LEAD) % NBUF)
                g_wait((b + 2) % NBUF)
                o_start(k, (b + 2) % NBUF)

        for k in range(ngroups * NBUF + 2, nchunk):
            step(k, k % NBUF)

        for b in range(NBUF):
            o_wait(b)

    return _kernel(table, idx3d)


def kernel(x, emb_weight):
    batch, hist = x.shape
    idx3d = x.reshape(_NW, -1, WINDOW).astype(jnp.int32)
    return _gather_rows(emb_weight, idx3d, batch)


# speedup vs baseline: 10.7844x; 1.8054x over previous
import jax
import jax.numpy as jnp
from jax import lax
from jax.experimental import pallas as pl
from jax.experimental.pallas import tpu as pltpu
from jax.experimental.pallas import tpu_sc as plsc

EMB_DIM = 128
BPW = 128
NBUF = 5
LEAD = 3

_vector_mesh = plsc.VectorSubcoreMesh(
    core_axis_name="core", subcore_axis_name="subcore"
)
_NW = 32


def _gather_rows(table, xt):
    hist, batch = xt.shape
    nchunk = hist

    @pl.kernel(
        out_type=jax.ShapeDtypeStruct((hist, batch, EMB_DIM), table.dtype),
        mesh=_vector_mesh,
        scratch_types=[
            pltpu.VMEM((hist, BPW), jnp.int32),
            pltpu.VMEM((NBUF, BPW, EMB_DIM), table.dtype),
            pltpu.SemaphoreType.DMA((NBUF,)),
            pltpu.SemaphoreType.DMA((NBUF,)),
        ],
    )
    def _kernel(table_hbm, xt_hbm, out_hbm, idx_v, bufs, gsem, osem):
        wid = lax.axis_index("subcore") * 2 + lax.axis_index("core")
        col_base = wid * BPW

        def g_start(k, b):
            pltpu.make_async_copy(
                table_hbm.at[idx_v.at[k]], bufs.at[b], gsem.at[b]
            ).start()

        def g_wait(b):
            pltpu.make_async_copy(
                table_hbm.at[idx_v.at[0]], bufs.at[b], gsem.at[b]
            ).wait()

        def o_start(k, b):
            pltpu.make_async_copy(
                bufs.at[b],
                out_hbm.at[k].at[pl.ds(col_base, BPW)],
                osem.at[b],
            ).start()

        def o_wait(b):
            pltpu.make_async_copy(
                bufs.at[b],
                out_hbm.at[0].at[pl.ds(0, BPW)],
                osem.at[b],
            ).wait()

        pltpu.sync_copy(xt_hbm.at[:, pl.ds(col_base, BPW)], idx_v)

        def step(k, kmod):
            if k + LEAD < nchunk:
                if k >= 2:
                    o_wait((kmod + LEAD) % NBUF)
                g_start(k + LEAD, (kmod + LEAD) % NBUF)
            g_wait(kmod % NBUF)
            o_start(k, kmod % NBUF)

        for b in range(LEAD):
            g_start(b, b)

        step(0, 0)
        step(1, 1)

        ngroups = (nchunk - LEAD - 2) // NBUF

        @pl.loop(0, ngroups)
        def _(g):
            for b in range(NBUF):
                k = g * NBUF + b + 2
                o_wait((b + 2 + LEAD) % NBUF)
                g_start(k + LEAD, (b + 2 + LEAD) % NBUF)
                g_wait((b + 2) % NBUF)
                o_start(k, (b + 2) % NBUF)

        for k in range(ngroups * NBUF + 2, nchunk):
            step(k, k % NBUF)

        for b in range(NBUF):
            o_wait(b)

    return _kernel(table, xt)


def kernel(x, emb_weight):
    batch, hist = x.shape
    xt = x.T.astype(jnp.int32)
    out3 = _gather_rows(emb_weight, xt)
    return jnp.transpose(out3, (1, 0, 2))


# speedup vs baseline: 10.7979x; 1.0013x over previous
import jax
import jax.numpy as jnp
from jax import lax
from jax.experimental import pallas as pl
from jax.experimental.pallas import tpu as pltpu
from jax.experimental.pallas import tpu_sc as plsc

EMB_DIM = 128
BPW = 128
NBUF = 6
LEAD = 4

_vector_mesh = plsc.VectorSubcoreMesh(
    core_axis_name="core", subcore_axis_name="subcore"
)
_NW = 32


def _gather_rows(table, xt):
    hist, batch = xt.shape
    nchunk = hist

    @pl.kernel(
        out_type=jax.ShapeDtypeStruct((hist, batch, EMB_DIM), table.dtype),
        mesh=_vector_mesh,
        scratch_types=[
            pltpu.VMEM((hist, BPW), jnp.int32),
            pltpu.VMEM((NBUF, BPW, EMB_DIM), table.dtype),
            pltpu.SemaphoreType.DMA((NBUF,)),
            pltpu.SemaphoreType.DMA((NBUF,)),
        ],
    )
    def _kernel(table_hbm, xt_hbm, out_hbm, idx_v, bufs, gsem, osem):
        wid = lax.axis_index("subcore") * 2 + lax.axis_index("core")
        col_base = wid * BPW

        def g_start(k, b):
            pltpu.make_async_copy(
                table_hbm.at[idx_v.at[k]], bufs.at[b], gsem.at[b]
            ).start()

        def g_wait(b):
            pltpu.make_async_copy(
                table_hbm.at[idx_v.at[0]], bufs.at[b], gsem.at[b]
            ).wait()

        def o_start(k, b):
            pltpu.make_async_copy(
                bufs.at[b],
                out_hbm.at[k].at[pl.ds(col_base, BPW)],
                osem.at[b],
            ).start()

        def o_wait(b):
            pltpu.make_async_copy(
                bufs.at[b],
                out_hbm.at[0].at[pl.ds(0, BPW)],
                osem.at[b],
            ).wait()

        pltpu.sync_copy(xt_hbm.at[:, pl.ds(col_base, BPW)], idx_v)

        def step(k, kmod):
            if k + LEAD < nchunk:
                if k >= 2:
                    o_wait((kmod + LEAD) % NBUF)
                g_start(k + LEAD, (kmod + LEAD) % NBUF)
            g_wait(kmod % NBUF)
            o_start(k, kmod % NBUF)

        for b in range(LEAD):
            g_start(b, b)

        step(0, 0)
        step(1, 1)

        ngroups = (nchunk - LEAD - 2) // NBUF

        @pl.loop(0, ngroups)
        def _(g):
            for b in range(NBUF):
                k = g * NBUF + b + 2
                o_wait((b + 2 + LEAD) % NBUF)
                g_start(k + LEAD, (b + 2 + LEAD) % NBUF)
                g_wait((b + 2) % NBUF)
                o_start(k, (b + 2) % NBUF)

        for k in range(ngroups * NBUF + 2, nchunk):
            step(k, k % NBUF)

        for b in range(NBUF):
            o_wait(b)

    return _kernel(table, xt)


def kernel(x, emb_weight):
    batch, hist = x.shape
    xt = x.T.astype(jnp.int32)
    out3 = _gather_rows(emb_weight, xt)
    return jnp.transpose(out3, (1, 0, 2))
